# Initial kernel scaffold; baseline (speedup 1.0000x reference)
import jax
import jax.numpy as jnp
from jax.experimental import pallas as pl


def kernel(Hx_dirs, edge_index_, bounds, delays, rate, numofbyte, bandwidth, W1, b1, W2, b2, pd_W, pd_b):
    raise NotImplementedError("write your pallas kernel here")


# speedup vs baseline: 1.0480x; 1.0480x over previous
import jax
import jax.numpy as jnp
from jax.experimental import pallas as pl
from jax.experimental.pallas import tpu as pltpu

NE = 2048
NN = 64
NKC = 16
NH = 64
EB = 128
GRID = NE // EB


def _mm_kernel(hx_ref, w1t_ref, b1_ref, w2_ref, bnd_ref, pdw_ref, pdb_ref,
               dly_ref, sc_ref, out_ref, lp_ref, acc_s, acc_a):
    i = pl.program_id(0)
    kf = float(NKC)

    f = hx_ref[:, :, NKC]
    x = hx_ref[:, :, NKC + 1]
    ft = f.T
    xt = x.T

    s = jnp.sum(ft * xt, axis=0, keepdims=True)
    agg = ft * s + (kf - 1.0) * ft * ft * xt

    w10 = w1t_ref[:, 0:1]
    w11 = w1t_ref[:, 1:2]
    b1 = b1_ref[:, 0:1]
    w2 = w2_ref[:, 0:1]

    ph = jnp.tanh(xt[:, None, :] * w10[None, :, :]
                  + agg[:, None, :] * w11[None, :, :]
                  + b1[None, :, :])
    b2v = sc_ref[:, 1:2]
    pt = jnp.sum(ph * w2[None, :, :], axis=1) + b2v

    pdw = pdw_ref[...]
    wsum = jnp.sum(pdw[:, :NKC, :], axis=1)
    pw16 = pdw[:, NKC, :]
    pw17 = pdw[:, NKC + 1, :]
    pdb = pdb_ref[...]

    b0 = bnd_ref[:, 0:1]
    b1c = bnd_ref[:, 1:2]
    lo = jnp.minimum(b0, b1c)
    hi = jnp.maximum(b0, b1c)

    raw = ((pt[:, None, :] * (1.0 / kf)) * wsum[:, :, None]
           + ft[:, None, :] * pw16[:, :, None]
           + xt[:, None, :] * pw17[:, :, None]
           + pdb[:, :, None])
    scale = sc_ref[:, 0:1]
    pts = (lo[:, :, None] + jax.nn.sigmoid(raw) * (hi - lo)[:, :, None]) \
        * scale[:, :, None]

    out_ref[...] = jnp.transpose(pts, (0, 2, 1))

    ps = jnp.sum(pts, axis=2)
    pa = jnp.sum(jnp.abs(pts), axis=2)

    @pl.when(i == 0)
    def _():
        acc_s[...] = ps
        acc_a[...] = pa

    @pl.when(i > 0)
    def _():
        acc_s[...] = acc_s[...] + ps
        acc_a[...] = acc_a[...] + pa

    @pl.when(i == GRID - 1)
    def _():
        inv = 1.0 / float(NE * NKC)
        dn = jnp.sum(acc_s[...], axis=1, keepdims=True) * inv
        ln = jnp.sum(acc_a[...], axis=1, keepdims=True) * inv
        delay = -jnp.sum(dn) / float(NN)
        sq = jnp.sum((ln + dly_ref[...]) ** 2) / float(NN - 1)
        lp_ref[...] = jnp.reshape(delay - sq, (1, 1))


def kernel(Hx_dirs, edge_index_, bounds, delays, rate, numofbyte, bandwidth,
           W1, b1, W2, b2, pd_W, pd_b):
    w1t = jnp.transpose(W1)
    b1c = jnp.reshape(b1, (NH, 1))
    w2c = jnp.reshape(W2, (NH, 1))
    dly = jnp.reshape(delays, (NN, 1))
    scale = rate[0] * jnp.asarray(numofbyte).astype(jnp.float32) \
        / (bandwidth[0] + 1.0)
    sc = jnp.stack([scale, b2[0]]).reshape(1, 2)

    out, lp = pl.pallas_call(
        _mm_kernel,
        grid=(GRID,),
        in_specs=[
            pl.BlockSpec((EB, NN, NKC + 2), lambda i: (i, 0, 0)),
            pl.BlockSpec((NH, 2), lambda i: (0, 0)),
            pl.BlockSpec((NH, 1), lambda i: (0, 0)),
            pl.BlockSpec((NH, 1), lambda i: (0, 0)),
            pl.BlockSpec((NN, 2), lambda i: (0, 0)),
            pl.BlockSpec((NN, NKC + 2, NKC), lambda i: (0, 0, 0)),
            pl.BlockSpec((NN, NKC), lambda i: (0, 0)),
            pl.BlockSpec((NN, 1), lambda i: (0, 0)),
            pl.BlockSpec((1, 2), lambda i: (0, 0)),
        ],
        out_specs=[
            pl.BlockSpec((NN, EB, NKC), lambda i: (0, i, 0)),
            pl.BlockSpec((1, 1), lambda i: (0, 0)),
        ],
        out_shape=[
            jax.ShapeDtypeStruct((NN, NE, NKC), jnp.float32),
            jax.ShapeDtypeStruct((1, 1), jnp.float32),
        ],
        scratch_shapes=[
            pltpu.VMEM((NN, NKC), jnp.float32),
            pltpu.VMEM((NN, NKC), jnp.float32),
        ],
    )(Hx_dirs, w1t, b1c, w2c, bounds, pd_W, pd_b, dly, sc)
    return out, jnp.reshape(lp, (1,))


# speedup vs baseline: 2.1514x; 2.0529x over previous
import jax
import jax.numpy as jnp
from jax.experimental import pallas as pl
from jax.experimental.pallas import tpu as pltpu

NE = 2048
NN = 64
NKC = 16
NH = 64
EB = 256
GRID = NE // EB


def _mm_kernel(ft_ref, xt_ref, w1t_ref, b1_ref, w2_ref, bnd_ref, pdw_ref,
               pdb_ref, dly_ref, sc_ref, out_ref, lp_ref, acc_s, acc_a):
    i = pl.program_id(0)
    kf = float(NKC)

    ft = ft_ref[...]
    xt = xt_ref[...]

    s = jnp.sum(ft * xt, axis=0, keepdims=True)
    agg = ft * s + (kf - 1.0) * ft * ft * xt

    w10 = w1t_ref[:, 0:1]
    w11 = w1t_ref[:, 1:2]
    b1 = b1_ref[:, 0:1]
    w2 = w2_ref[:, 0:1]

    ph = jnp.tanh(xt[:, None, :] * w10[None, :, :]
                  + agg[:, None, :] * w11[None, :, :]
                  + b1[None, :, :])
    b2v = sc_ref[:, 1:2]
    pt = jnp.sum(ph * w2[None, :, :], axis=1) + b2v

    pdw = pdw_ref[...]
    wsum = jnp.sum(pdw[:, :NKC, :], axis=1)
    pw16 = pdw[:, NKC, :]
    pw17 = pdw[:, NKC + 1, :]
    pdb = pdb_ref[...]

    b0 = bnd_ref[:, 0:1]
    b1c = bnd_ref[:, 1:2]
    lo = jnp.minimum(b0, b1c)
    hi = jnp.maximum(b0, b1c)

    raw = ((pt[:, None, :] * (1.0 / kf)) * wsum[:, :, None]
           + ft[:, None, :] * pw16[:, :, None]
           + xt[:, None, :] * pw17[:, :, None]
           + pdb[:, :, None])
    scale = sc_ref[:, 0:1]
    pts = (lo[:, :, None] + jax.nn.sigmoid(raw) * (hi - lo)[:, :, None]) \
        * scale[:, :, None]

    out_ref[...] = jnp.transpose(pts, (0, 2, 1))

    ps = jnp.sum(pts, axis=2)
    pa = jnp.sum(jnp.abs(pts), axis=2)

    @pl.when(i == 0)
    def _():
        acc_s[...] = ps
        acc_a[...] = pa

    @pl.when(i > 0)
    def _():
        acc_s[...] = acc_s[...] + ps
        acc_a[...] = acc_a[...] + pa

    @pl.when(i == GRID - 1)
    def _():
        inv = 1.0 / float(NE * NKC)
        dn = jnp.sum(acc_s[...], axis=1, keepdims=True) * inv
        ln = jnp.sum(acc_a[...], axis=1, keepdims=True) * inv
        delay = -jnp.sum(dn) / float(NN)
        sq = jnp.sum((ln + dly_ref[...]) ** 2) / float(NN - 1)
        lp_ref[...] = jnp.reshape(delay - sq, (1, 1))


def kernel(Hx_dirs, edge_index_, bounds, delays, rate, numofbyte, bandwidth,
           W1, b1, W2, b2, pd_W, pd_b):
    w1t = jnp.transpose(W1)
    b1c = jnp.reshape(b1, (NH, 1))
    w2c = jnp.reshape(W2, (NH, 1))
    dly = jnp.reshape(delays, (NN, 1))
    scale = rate[0] * jnp.asarray(numofbyte).astype(jnp.float32) \
        / (bandwidth[0] + 1.0)
    sc = jnp.stack([scale, b2[0]]).reshape(1, 2)
    fT = jnp.transpose(Hx_dirs[:, :, NKC])
    xT = jnp.transpose(Hx_dirs[:, :, NKC + 1])

    out, lp = pl.pallas_call(
        _mm_kernel,
        grid=(GRID,),
        in_specs=[
            pl.BlockSpec((NN, EB), lambda i: (0, i)),
            pl.BlockSpec((NN, EB), lambda i: (0, i)),
            pl.BlockSpec((NH, 2), lambda i: (0, 0)),
            pl.BlockSpec((NH, 1), lambda i: (0, 0)),
            pl.BlockSpec((NH, 1), lambda i: (0, 0)),
            pl.BlockSpec((NN, 2), lambda i: (0, 0)),
            pl.BlockSpec((NN, NKC + 2, NKC), lambda i: (0, 0, 0)),
            pl.BlockSpec((NN, NKC), lambda i: (0, 0)),
            pl.BlockSpec((NN, 1), lambda i: (0, 0)),
            pl.BlockSpec((1, 2), lambda i: (0, 0)),
        ],
        out_specs=[
            pl.BlockSpec((NN, EB, NKC), lambda i: (0, i, 0)),
            pl.BlockSpec((1, 1), lambda i: (0, 0)),
        ],
        out_shape=[
            jax.ShapeDtypeStruct((NN, NE, NKC), jnp.float32),
            jax.ShapeDtypeStruct((1, 1), jnp.float32),
        ],
        scratch_shapes=[
            pltpu.VMEM((NN, NKC), jnp.float32),
            pltpu.VMEM((NN, NKC), jnp.float32),
        ],
    )(fT, xT, w1t, b1c, w2c, bounds, pd_W, pd_b, dly, sc)
    return out, jnp.reshape(lp, (1,))


# speedup vs baseline: 2.1656x; 1.0066x over previous
import jax
import jax.numpy as jnp
from jax.experimental import pallas as pl
from jax.experimental.pallas import tpu as pltpu

NE = 2048
NN = 64
NKC = 16
NH = 64
EB = 256
GRID = NE // EB


def _mm_kernel(ft_ref, xt_ref, w1t_ref, b1_ref, w2_ref, bnd_ref, pdw_ref,
               pdb_ref, dly_ref, sc_ref, out_ref, lp_ref, acc_s, acc_a):
    i = pl.program_id(0)
    kf = float(NKC)

    ft = ft_ref[...].T
    xt = xt_ref[...].T

    s = jnp.sum(ft * xt, axis=0, keepdims=True)
    agg = ft * s + (kf - 1.0) * ft * ft * xt

    w10 = w1t_ref[:, 0:1]
    w11 = w1t_ref[:, 1:2]
    b1 = b1_ref[:, 0:1]
    w2 = w2_ref[:, 0:1]

    ph = jnp.tanh(xt[:, None, :] * w10[None, :, :]
                  + agg[:, None, :] * w11[None, :, :]
                  + b1[None, :, :])
    b2v = sc_ref[:, 1:2]
    pt = jnp.sum(ph * w2[None, :, :], axis=1) + b2v

    pdw = pdw_ref[...]
    wsum = jnp.sum(pdw[:, :NKC, :], axis=1)
    pw16 = pdw[:, NKC, :]
    pw17 = pdw[:, NKC + 1, :]
    pdb = pdb_ref[...]

    b0 = bnd_ref[:, 0:1]
    b1c = bnd_ref[:, 1:2]
    lo = jnp.minimum(b0, b1c)
    hi = jnp.maximum(b0, b1c)

    raw = ((pt[:, None, :] * (1.0 / kf)) * wsum[:, :, None]
           + ft[:, None, :] * pw16[:, :, None]
           + xt[:, None, :] * pw17[:, :, None]
           + pdb[:, :, None])
    scale = sc_ref[:, 0:1]
    pts = (lo[:, :, None] + jax.nn.sigmoid(raw) * (hi - lo)[:, :, None]) \
        * scale[:, :, None]

    out_ref[...] = jnp.transpose(pts, (0, 2, 1))

    ps = jnp.sum(pts, axis=2)
    pa = jnp.sum(jnp.abs(pts), axis=2)

    @pl.when(i == 0)
    def _():
        acc_s[...] = ps
        acc_a[...] = pa

    @pl.when(i > 0)
    def _():
        acc_s[...] = acc_s[...] + ps
        acc_a[...] = acc_a[...] + pa

    @pl.when(i == GRID - 1)
    def _():
        inv = 1.0 / float(NE * NKC)
        dn = jnp.sum(acc_s[...], axis=1, keepdims=True) * inv
        ln = jnp.sum(acc_a[...], axis=1, keepdims=True) * inv
        delay = -jnp.sum(dn) / float(NN)
        sq = jnp.sum((ln + dly_ref[...]) ** 2) / float(NN - 1)
        lp_ref[...] = jnp.reshape(delay - sq, (1, 1))


def kernel(Hx_dirs, edge_index_, bounds, delays, rate, numofbyte, bandwidth,
           W1, b1, W2, b2, pd_W, pd_b):
    w1t = jnp.transpose(W1)
    b1c = jnp.reshape(b1, (NH, 1))
    w2c = jnp.reshape(W2, (NH, 1))
    dly = jnp.reshape(delays, (NN, 1))
    scale = rate[0] * jnp.asarray(numofbyte).astype(jnp.float32) \
        / (bandwidth[0] + 1.0)
    sc = jnp.stack([scale, b2[0]]).reshape(1, 2)
    fT = Hx_dirs[:, :, NKC]
    xT = Hx_dirs[:, :, NKC + 1]

    out, lp = pl.pallas_call(
        _mm_kernel,
        grid=(GRID,),
        in_specs=[
            pl.BlockSpec((EB, NN), lambda i: (i, 0)),
            pl.BlockSpec((EB, NN), lambda i: (i, 0)),
            pl.BlockSpec((NH, 2), lambda i: (0, 0)),
            pl.BlockSpec((NH, 1), lambda i: (0, 0)),
            pl.BlockSpec((NH, 1), lambda i: (0, 0)),
            pl.BlockSpec((NN, 2), lambda i: (0, 0)),
            pl.BlockSpec((NN, NKC + 2, NKC), lambda i: (0, 0, 0)),
            pl.BlockSpec((NN, NKC), lambda i: (0, 0)),
            pl.BlockSpec((NN, 1), lambda i: (0, 0)),
            pl.BlockSpec((1, 2), lambda i: (0, 0)),
        ],
        out_specs=[
            pl.BlockSpec((NN, EB, NKC), lambda i: (0, i, 0)),
            pl.BlockSpec((1, 1), lambda i: (0, 0)),
        ],
        out_shape=[
            jax.ShapeDtypeStruct((NN, NE, NKC), jnp.float32),
            jax.ShapeDtypeStruct((1, 1), jnp.float32),
        ],
        scratch_shapes=[
            pltpu.VMEM((NN, NKC), jnp.float32),
            pltpu.VMEM((NN, NKC), jnp.float32),
        ],
    )(fT, xT, w1t, b1c, w2c, bounds, pd_W, pd_b, dly, sc)
    return out, jnp.reshape(lp, (1,))


# speedup vs baseline: 4.9308x; 2.2769x over previous
import jax
import jax.numpy as jnp
from jax.experimental import pallas as pl
from jax.experimental.pallas import tpu as pltpu

NE = 2048
NN = 64
NKC = 16
NH = 64
EB = 256
GRID = NE // EB


def _mm_kernel(ft_ref, xt_ref, w1t_ref, b1_ref, w2_ref, bnd_ref, pdw_ref,
               pdb_ref, dly_ref, sc_ref, out_ref, lp_ref, acc_s, acc_a):
    i = pl.program_id(0)
    kf = float(NKC)

    ft = ft_ref[...].T
    xt = xt_ref[...].T

    s = jnp.sum(ft * xt, axis=0, keepdims=True)
    agg = ft * s + (kf - 1.0) * ft * ft * xt

    w10 = w1t_ref[:, 0:1]
    w11 = w1t_ref[:, 1:2]
    b1 = b1_ref[:, 0:1]
    w2 = w2_ref[:, 0:1]

    ph = jnp.tanh(xt[:, None, :] * w10[None, :, :]
                  + agg[:, None, :] * w11[None, :, :]
                  + b1[None, :, :])
    b2v = sc_ref[:, 1:2]
    pt = jnp.sum(ph * w2[None, :, :], axis=1) + b2v

    pdw = pdw_ref[...]
    wsum = jnp.sum(pdw[:, :NKC, :], axis=1)
    pw16 = pdw[:, NKC, :]
    pw17 = pdw[:, NKC + 1, :]
    pdb = pdb_ref[...]

    b0 = bnd_ref[:, 0:1]
    b1c = bnd_ref[:, 1:2]
    lo = jnp.minimum(b0, b1c)
    hi = jnp.maximum(b0, b1c)

    raw = ((pt[:, None, :] * (1.0 / kf)) * wsum[:, :, None]
           + ft[:, None, :] * pw16[:, :, None]
           + xt[:, None, :] * pw17[:, :, None]
           + pdb[:, :, None])
    scale = sc_ref[:, 0:1]
    pts = (lo[:, :, None] + jax.nn.sigmoid(raw) * (hi - lo)[:, :, None]) \
        * scale[:, :, None]

    out_ref[...] = pts

    ps = jnp.sum(pts, axis=2)
    pa = jnp.sum(jnp.abs(pts), axis=2)

    @pl.when(i == 0)
    def _():
        acc_s[...] = ps
        acc_a[...] = pa

    @pl.when(i > 0)
    def _():
        acc_s[...] = acc_s[...] + ps
        acc_a[...] = acc_a[...] + pa

    @pl.when(i == GRID - 1)
    def _():
        inv = 1.0 / float(NE * NKC)
        dn = jnp.sum(acc_s[...], axis=1, keepdims=True) * inv
        ln = jnp.sum(acc_a[...], axis=1, keepdims=True) * inv
        delay = -jnp.sum(dn) / float(NN)
        sq = jnp.sum((ln + dly_ref[...]) ** 2) / float(NN - 1)
        lp_ref[...] = jnp.reshape(delay - sq, (1, 1))


def kernel(Hx_dirs, edge_index_, bounds, delays, rate, numofbyte, bandwidth,
           W1, b1, W2, b2, pd_W, pd_b):
    w1t = jnp.transpose(W1)
    b1c = jnp.reshape(b1, (NH, 1))
    w2c = jnp.reshape(W2, (NH, 1))
    dly = jnp.reshape(delays, (NN, 1))
    scale = rate[0] * jnp.asarray(numofbyte).astype(jnp.float32) \
        / (bandwidth[0] + 1.0)
    sc = jnp.stack([scale, b2[0]]).reshape(1, 2)
    fT = Hx_dirs[:, :, NKC]
    xT = Hx_dirs[:, :, NKC + 1]

    out, lp = pl.pallas_call(
        _mm_kernel,
        grid=(GRID,),
        in_specs=[
            pl.BlockSpec((EB, NN), lambda i: (i, 0)),
            pl.BlockSpec((EB, NN), lambda i: (i, 0)),
            pl.BlockSpec((NH, 2), lambda i: (0, 0)),
            pl.BlockSpec((NH, 1), lambda i: (0, 0)),
            pl.BlockSpec((NH, 1), lambda i: (0, 0)),
            pl.BlockSpec((NN, 2), lambda i: (0, 0)),
            pl.BlockSpec((NN, NKC + 2, NKC), lambda i: (0, 0, 0)),
            pl.BlockSpec((NN, NKC), lambda i: (0, 0)),
            pl.BlockSpec((NN, 1), lambda i: (0, 0)),
            pl.BlockSpec((1, 2), lambda i: (0, 0)),
        ],
        out_specs=[
            pl.BlockSpec((NN, NKC, EB), lambda i: (0, 0, i)),
            pl.BlockSpec((1, 1), lambda i: (0, 0)),
        ],
        out_shape=[
            jax.ShapeDtypeStruct((NN, NKC, NE), jnp.float32),
            jax.ShapeDtypeStruct((1, 1), jnp.float32),
        ],
        scratch_shapes=[
            pltpu.VMEM((NN, NKC), jnp.float32),
            pltpu.VMEM((NN, NKC), jnp.float32),
        ],
    )(fT, xT, w1t, b1c, w2c, bounds, pd_W, pd_b, dly, sc)
    return jnp.transpose(out, (0, 2, 1)), jnp.reshape(lp, (1,))


# speedup vs baseline: 5.2736x; 1.0695x over previous
import jax
import jax.numpy as jnp
from jax.experimental import pallas as pl
from jax.experimental.pallas import tpu as pltpu

NE = 2048
NN = 64
NKC = 16
NH = 64
EB = 256
GRID = NE // EB


def _mm_kernel(fx_ref, w1t_ref, b1_ref, w2_ref, bnd_ref, pdw_ref,
               pdb_ref, dly_ref, sc_ref, out_ref, lp_ref, acc_s, acc_a):
    i = pl.program_id(0)
    kf = float(NKC)

    ft = fx_ref[0]
    xt = fx_ref[1]

    s = jnp.sum(ft * xt, axis=0, keepdims=True)
    agg = ft * s + (kf - 1.0) * ft * ft * xt

    w10 = w1t_ref[:, 0:1]
    w11 = w1t_ref[:, 1:2]
    b1 = b1_ref[:, 0:1]
    w2 = w2_ref[:, 0:1]

    ph = jnp.tanh(xt[:, None, :] * w10[None, :, :]
                  + agg[:, None, :] * w11[None, :, :]
                  + b1[None, :, :])
    b2v = sc_ref[:, 1:2]
    pt = jnp.sum(ph * w2[None, :, :], axis=1) + b2v

    pdw = pdw_ref[...]
    wsum = jnp.sum(pdw[:, :NKC, :], axis=1)
    pw16 = pdw[:, NKC, :]
    pw17 = pdw[:, NKC + 1, :]
    pdb = pdb_ref[...]

    b0 = bnd_ref[:, 0:1]
    b1c = bnd_ref[:, 1:2]
    lo = jnp.minimum(b0, b1c)
    hi = jnp.maximum(b0, b1c)

    raw = ((pt[:, None, :] * (1.0 / kf)) * wsum[:, :, None]
           + ft[:, None, :] * pw16[:, :, None]
           + xt[:, None, :] * pw17[:, :, None]
           + pdb[:, :, None])
    scale = sc_ref[:, 0:1]
    pts = (lo[:, :, None] + jax.nn.sigmoid(raw) * (hi - lo)[:, :, None]) \
        * scale[:, :, None]

    out_ref[...] = pts

    ps = jnp.sum(pts, axis=2)
    pa = jnp.sum(jnp.abs(pts), axis=2)

    @pl.when(i == 0)
    def _():
        acc_s[...] = ps
        acc_a[...] = pa

    @pl.when(i > 0)
    def _():
        acc_s[...] = acc_s[...] + ps
        acc_a[...] = acc_a[...] + pa

    @pl.when(i == GRID - 1)
    def _():
        inv = 1.0 / float(NE * NKC)
        dn = jnp.sum(acc_s[...], axis=1, keepdims=True) * inv
        ln = jnp.sum(acc_a[...], axis=1, keepdims=True) * inv
        delay = -jnp.sum(dn) / float(NN)
        sq = jnp.sum((ln + dly_ref[...]) ** 2) / float(NN - 1)
        lp_ref[...] = jnp.reshape(delay - sq, (1, 1))


def kernel(Hx_dirs, edge_index_, bounds, delays, rate, numofbyte, bandwidth,
           W1, b1, W2, b2, pd_W, pd_b):
    w1t = jnp.transpose(W1)
    b1c = jnp.reshape(b1, (NH, 1))
    w2c = jnp.reshape(W2, (NH, 1))
    dly = jnp.reshape(delays, (NN, 1))
    scale = rate[0] * jnp.asarray(numofbyte).astype(jnp.float32) \
        / (bandwidth[0] + 1.0)
    sc = jnp.stack([scale, b2[0]]).reshape(1, 2)
    fxT = jnp.transpose(Hx_dirs[:, :, NKC:NKC + 2], (2, 1, 0))

    out, lp = pl.pallas_call(
        _mm_kernel,
        grid=(GRID,),
        in_specs=[
            pl.BlockSpec((2, NN, EB), lambda i: (0, 0, i)),
            pl.BlockSpec((NH, 2), lambda i: (0, 0)),
            pl.BlockSpec((NH, 1), lambda i: (0, 0)),
            pl.BlockSpec((NH, 1), lambda i: (0, 0)),
            pl.BlockSpec((NN, 2), lambda i: (0, 0)),
            pl.BlockSpec((NN, NKC + 2, NKC), lambda i: (0, 0, 0)),
            pl.BlockSpec((NN, NKC), lambda i: (0, 0)),
            pl.BlockSpec((NN, 1), lambda i: (0, 0)),
            pl.BlockSpec((1, 2), lambda i: (0, 0)),
        ],
        out_specs=[
            pl.BlockSpec((NN, NKC, EB), lambda i: (0, 0, i)),
            pl.BlockSpec((1, 1), lambda i: (0, 0)),
        ],
        out_shape=[
            jax.ShapeDtypeStruct((NN, NKC, NE), jnp.float32),
            jax.ShapeDtypeStruct((1, 1), jnp.float32),
        ],
        scratch_shapes=[
            pltpu.VMEM((NN, NKC), jnp.float32),
            pltpu.VMEM((NN, NKC), jnp.float32),
        ],
    )(fxT, w1t, b1c, w2c, bounds, pd_W, pd_b, dly, sc)
    return jnp.transpose(out, (0, 2, 1)), jnp.reshape(lp, (1,))
